# baseline (speedup 1.0000x reference)
import jax
import jax.numpy as jnp
from jax.experimental import pallas as pl
from jax.experimental.pallas import tpu as pltpu

B, D, C = 2048, 1024, 8192
BB = 1024
BC = 1024

_LN2 = 0.6931471805599453
_INV_LN2 = 1.4426950408889634


def _fused_kernel(x_ref, ct_ref, y_ref, loss_ref, corr_ref,
                  xs_ref, m_ref, l_ref, t_ref, a_ref):
    c = pl.program_id(1)
    nc = pl.num_programs(1)

    @pl.when(c == 0)
    def _init():
        xs_ref[...] = x_ref[...] * (2.0 * _INV_LN2)
        m_ref[...] = jnp.full(m_ref.shape, -jnp.inf, dtype=jnp.float32)
        l_ref[...] = jnp.zeros(l_ref.shape, dtype=jnp.float32)
        t_ref[...] = jnp.zeros(t_ref.shape, dtype=jnp.float32)
        a_ref[...] = jnp.zeros(a_ref.shape, dtype=jnp.float32)

    ctb = ct_ref[...]
    acc = jnp.dot(xs_ref[...], ctb, preferred_element_type=jnp.float32)
    c2h = jnp.sum(ctb * ctb, axis=0, keepdims=True) * _INV_LN2
    h = acc - c2h

    cmax = jnp.max(h, axis=1, keepdims=True)
    camax = jnp.argmax(h, axis=1, keepdims=True).astype(jnp.float32) + c * BC
    col = jax.lax.broadcasted_iota(jnp.int32, (BB, BC), 1)
    yloc = y_ref[...] - c * BC
    tsum = jnp.sum(jnp.where(col == yloc, h, 0.0),
                   axis=1, keepdims=True)

    m_old = jnp.max(m_ref[...], axis=1, keepdims=True)
    l_old = jnp.max(l_ref[...], axis=1, keepdims=True)
    a_old = jnp.max(a_ref[...], axis=1, keepdims=True)

    m_new = jnp.maximum(m_old, cmax)
    p_sum = jnp.sum(jnp.exp2(h - m_new), axis=1, keepdims=True)
    l_new = l_old * jnp.exp2(m_old - m_new) + p_sum
    a_new = jnp.where(cmax > m_old, camax, a_old)

    m_ref[...] = jnp.broadcast_to(m_new, m_ref.shape)
    l_ref[...] = jnp.broadcast_to(l_new, l_ref.shape)
    a_ref[...] = jnp.broadcast_to(a_new, a_ref.shape)
    t_ref[...] = t_ref[...] + jnp.broadcast_to(tsum, t_ref.shape)

    @pl.when(c == nc - 1)
    def _fin():
        m_c = jnp.max(m_ref[...], axis=1, keepdims=True)
        l_c = jnp.max(l_ref[...], axis=1, keepdims=True)
        t_c = jnp.max(t_ref[...], axis=1, keepdims=True)
        a_c = jnp.max(a_ref[...], axis=1, keepdims=True)
        loss_col = (m_c + jnp.log2(l_c) - t_c) * _LN2
        corr_col = (a_c == y_ref[...].astype(jnp.float32)).astype(jnp.float32)
        ls = jnp.sum(loss_col, keepdims=True)
        cs = jnp.sum(corr_col, keepdims=True)
        loss_ref[...] = jnp.broadcast_to(ls, (8, 128)).reshape(1, 8, 128)
        corr_ref[...] = jnp.broadcast_to(cs, (8, 128)).reshape(1, 8, 128)


@jax.jit
def kernel(x, y, centroids):
    ct = centroids.T
    y_col = y.astype(jnp.int32).reshape(B, 1)
    nb = B // BB
    out_shape = (jax.ShapeDtypeStruct((nb, 8, 128), jnp.float32),
                 jax.ShapeDtypeStruct((nb, 8, 128), jnp.float32))
    loss_p, corr_p = pl.pallas_call(
        _fused_kernel,
        grid=(nb, C // BC),
        in_specs=[
            pl.BlockSpec((BB, D), lambda b, c: (b, 0)),
            pl.BlockSpec((D, BC), lambda b, c: (0, c)),
            pl.BlockSpec((BB, 1), lambda b, c: (b, 0)),
        ],
        out_specs=(pl.BlockSpec((1, 8, 128), lambda b, c: (b, 0, 0)),
                   pl.BlockSpec((1, 8, 128), lambda b, c: (b, 0, 0))),
        out_shape=out_shape,
        scratch_shapes=[
            pltpu.VMEM((BB, D), jnp.float32),
            pltpu.VMEM((BB, 128), jnp.float32),
            pltpu.VMEM((BB, 128), jnp.float32),
            pltpu.VMEM((BB, 128), jnp.float32),
            pltpu.VMEM((BB, 128), jnp.float32),
        ],
        compiler_params=pltpu.CompilerParams(
            dimension_semantics=("parallel", "arbitrary"),
            vmem_limit_bytes=100 * 1024 * 1024,
        ),
    )(x, ct, y_col)
    loss = jnp.sum(loss_p[:, 0, 0]) / B
    score = jnp.sum(corr_p[:, 0, 0]) / B
    return loss, score


# speedup vs baseline: 1.4915x; 1.4915x over previous
import jax
import jax.numpy as jnp
from jax.experimental import pallas as pl
from jax.experimental.pallas import tpu as pltpu

B, D, C = 2048, 1024, 8192
BB = 1024
BC = 1024

_LN2 = 0.6931471805599453
_INV_LN2 = 1.4426950408889634


def _fused_kernel(xt_ref, cen_ref, y_ref, loss_ref, corr_ref,
                  xs_ref, m_ref, l_ref, t_ref, a_ref):
    c = pl.program_id(1)
    nc = pl.num_programs(1)

    @pl.when(c == 0)
    def _init():
        xs_ref[...] = xt_ref[...] * (2.0 * _INV_LN2)
        m_ref[...] = jnp.full(m_ref.shape, -jnp.inf, dtype=jnp.float32)
        l_ref[...] = jnp.zeros(l_ref.shape, dtype=jnp.float32)
        t_ref[...] = jnp.zeros(t_ref.shape, dtype=jnp.float32)
        a_ref[...] = jnp.zeros(a_ref.shape, dtype=jnp.float32)

    cb = cen_ref[...]
    acc = jnp.dot(cb, xs_ref[...], preferred_element_type=jnp.float32)
    c2h = jnp.sum(cb * cb, axis=1, keepdims=True) * _INV_LN2
    h = acc - c2h

    cmax = jnp.max(h, axis=0, keepdims=True)
    row = jax.lax.broadcasted_iota(jnp.int32, (BC, BB), 0)
    camax = jnp.min(jnp.where(h >= cmax, row, C), axis=0,
                    keepdims=True).astype(jnp.float32) + c * BC
    yloc = y_ref[0] - c * BC
    tsum = jnp.sum(jnp.where(row == yloc, h, 0.0),
                   axis=0, keepdims=True)

    m_old = jnp.max(m_ref[...], axis=0, keepdims=True)
    l_old = jnp.max(l_ref[...], axis=0, keepdims=True)
    a_old = jnp.max(a_ref[...], axis=0, keepdims=True)

    m_new = jnp.maximum(m_old, cmax)
    p_sum = jnp.sum(jnp.exp2(h - m_new), axis=0, keepdims=True)
    l_new = l_old * jnp.exp2(m_old - m_new) + p_sum
    a_new = jnp.where(cmax > m_old, camax, a_old)

    m_ref[...] = jnp.broadcast_to(m_new, m_ref.shape)
    l_ref[...] = jnp.broadcast_to(l_new, l_ref.shape)
    a_ref[...] = jnp.broadcast_to(a_new, a_ref.shape)
    t_ref[...] = t_ref[...] + jnp.broadcast_to(tsum, t_ref.shape)

    @pl.when(c == nc - 1)
    def _fin():
        m_c = jnp.max(m_ref[...], axis=0, keepdims=True)
        l_c = jnp.max(l_ref[...], axis=0, keepdims=True)
        t_c = jnp.max(t_ref[...], axis=0, keepdims=True)
        a_c = jnp.max(a_ref[...], axis=0, keepdims=True)
        loss_row = (m_c + jnp.log2(l_c) - t_c) * _LN2
        corr_row = (a_c == y_ref[0].astype(jnp.float32)).astype(jnp.float32)
        ls = jnp.sum(loss_row, keepdims=True)
        cs = jnp.sum(corr_row, keepdims=True)
        loss_ref[...] = jnp.broadcast_to(ls, (8, 128)).reshape(1, 8, 128)
        corr_ref[...] = jnp.broadcast_to(cs, (8, 128)).reshape(1, 8, 128)


@jax.jit
def kernel(x, y, centroids):
    xt = x.T
    nb = B // BB
    y3 = y.astype(jnp.int32).reshape(nb, 1, BB)
    out_shape = (jax.ShapeDtypeStruct((nb, 8, 128), jnp.float32),
                 jax.ShapeDtypeStruct((nb, 8, 128), jnp.float32))
    loss_p, corr_p = pl.pallas_call(
        _fused_kernel,
        grid=(nb, C // BC),
        in_specs=[
            pl.BlockSpec((D, BB), lambda b, c: (0, b)),
            pl.BlockSpec((BC, D), lambda b, c: (c, 0)),
            pl.BlockSpec((1, 1, BB), lambda b, c: (b, 0, 0)),
        ],
        out_specs=(pl.BlockSpec((1, 8, 128), lambda b, c: (b, 0, 0)),
                   pl.BlockSpec((1, 8, 128), lambda b, c: (b, 0, 0))),
        out_shape=out_shape,
        scratch_shapes=[
            pltpu.VMEM((D, BB), jnp.float32),
            pltpu.VMEM((8, BB), jnp.float32),
            pltpu.VMEM((8, BB), jnp.float32),
            pltpu.VMEM((8, BB), jnp.float32),
            pltpu.VMEM((8, BB), jnp.float32),
        ],
        compiler_params=pltpu.CompilerParams(
            dimension_semantics=("parallel", "arbitrary"),
            vmem_limit_bytes=100 * 1024 * 1024,
        ),
    )(xt, centroids, y3)
    loss = jnp.sum(loss_p[:, 0, 0]) / B
    score = jnp.sum(corr_p[:, 0, 0]) / B
    return loss, score
